# baseline (speedup 1.0000x reference)
import jax
import jax.numpy as jnp
from jax.experimental import pallas as pl
from jax.experimental.pallas import tpu as pltpu

B, NQ, NK = 1, 2048, 1024
D_MSG, D_FF, D_OUT = 16, 32, 16
TQ = 128

_DN0 = (((0,), (0,)), ((), ()))
_DN1 = (((1,), (0,)), ((), ()))
_DN01 = (((0,), (1,)), ((), ()))


def _pair_body(qet_ref, qit_ref, kft_ref, kit_ref, wqt_ref, wkt_ref,
               w1t_ref, w2t_ref, b1c_ref, bqr_ref, bkc_ref, b2r_ref,
               o_ref, kx_ref):
    f32, bf16 = jnp.float32, jnp.bfloat16

    @pl.when(pl.program_id(0) == 0)
    def _build_kx():
        kmt = jax.lax.dot_general(wkt_ref[...], kit_ref[...], _DN1,
                                  preferred_element_type=f32)
        kx_ref[0:24, :] = kft_ref[...].astype(bf16)
        kx_ref[24:40, :] = (kmt + bkc_ref[...]).astype(bf16)
        kx_ref[40:57, :] = jnp.ones((17, NK), bf16)
        kx_ref[57:64, :] = jnp.zeros((7, NK), bf16)

    eye24 = (jax.lax.broadcasted_iota(jnp.int32, (24, 24), 0) ==
             jax.lax.broadcasted_iota(jnp.int32, (24, 24), 1)
             ).astype(jnp.bfloat16)
    qf = jax.lax.dot_general(qet_ref[...].astype(bf16), eye24, _DN0,
                             preferred_element_type=f32)
    qm = jax.lax.dot_general(qit_ref[...], wqt_ref[...], _DN01,
                             preferred_element_type=f32)
    qm = qm + bqr_ref[...]
    a = jnp.concatenate(
        [qf, jnp.ones((TQ, D_MSG), f32), qm, jnp.ones((TQ, 1), f32),
         jnp.zeros((TQ, 7), f32)], axis=1)

    w1t = w1t_ref[...]
    w1d_t = w1t[:, 32:40]
    bt = jnp.concatenate(
        [w1d_t, w1d_t, w1d_t, w1t[:, 16:32], w1t[:, 0:16],
         b1c_ref[...], jnp.zeros((D_FF, 7), f32)], axis=1)

    qw = (a[:, None, :] * bt[None, :, :]).astype(bf16).reshape(
        TQ * D_FF, 64)
    h = jax.lax.dot_general(qw, kx_ref[...], _DN1,
                            preferred_element_type=f32)
    u = (h * 0.5).astype(bf16)
    t = jnp.tanh(u)
    sb = u * t + u

    w2t = w2t_ref[...]
    row16 = jnp.concatenate([w2t] * 8, axis=1)
    tiled = jnp.concatenate([row16] * 8, axis=0)
    blk_mask = (jax.lax.broadcasted_iota(jnp.int32, (128, 256), 0) // 16
                == jax.lax.broadcasted_iota(jnp.int32, (128, 256), 1)
                // 32)
    wbd = jnp.where(blk_mask, tiled, 0.0).astype(bf16)

    b2_3d = b2r_ref[...][:, :, None]
    for g in range(TQ // 8):
        r = jax.lax.dot_general(wbd, sb[g * 256:(g + 1) * 256, :],
                                _DN1, preferred_element_type=f32)
        o_ref[g * 8:(g + 1) * 8, :, :] = r.reshape(8, D_OUT, NK) + b2_3d


def kernel(q_equi, q_inv, k_equi, k_inv, Wq, bq, Wk, bk, W1, b1, W2, b2):
    f32 = jnp.float32
    qet = q_equi.reshape(NQ, 24).T
    qit = q_inv.reshape(NQ, -1).T
    kft = k_equi.reshape(NK, 24).T
    kit = k_inv.reshape(NK, -1).T

    out_t = pl.pallas_call(
        _pair_body,
        grid=(NQ // TQ,),
        in_specs=[
            pl.BlockSpec((24, TQ), lambda i: (0, i)),
            pl.BlockSpec((64, TQ), lambda i: (0, i)),
            pl.BlockSpec((24, NK), lambda i: (0, 0)),
            pl.BlockSpec((64, NK), lambda i: (0, 0)),
            pl.BlockSpec((D_MSG, 64), lambda i: (0, 0)),
            pl.BlockSpec((D_MSG, 64), lambda i: (0, 0)),
            pl.BlockSpec((D_FF, 40), lambda i: (0, 0)),
            pl.BlockSpec((D_MSG, D_FF), lambda i: (0, 0)),
            pl.BlockSpec((D_FF, 1), lambda i: (0, 0)),
            pl.BlockSpec((1, D_MSG), lambda i: (0, 0)),
            pl.BlockSpec((D_MSG, 1), lambda i: (0, 0)),
            pl.BlockSpec((1, D_OUT), lambda i: (0, 0)),
        ],
        out_specs=pl.BlockSpec((TQ, D_OUT, NK), lambda i: (i, 0, 0)),
        out_shape=jax.ShapeDtypeStruct((NQ, D_OUT, NK), f32),
        scratch_shapes=[pltpu.VMEM((64, NK), jnp.bfloat16)],
    )(qet, qit, kft, kit, Wq.T, Wk.T, W1.T, W2.T,
      b1[:, None], bq[None, :], bk[:, None], b2[None, :])

    return out_t.reshape(B, NQ, D_OUT, NK).transpose(0, 1, 3, 2)


# speedup vs baseline: 1.0813x; 1.0813x over previous
import jax
import jax.numpy as jnp
from jax.experimental import pallas as pl
from jax.experimental.pallas import tpu as pltpu

B, NQ, NK = 1, 2048, 1024
D_MSG, D_FF, D_OUT = 16, 32, 16
TQ = 128

_DN0 = (((0,), (0,)), ((), ()))
_DN1 = (((1,), (0,)), ((), ()))
_DN01 = (((0,), (1,)), ((), ()))


def _pair_body(qet_ref, qit_ref, kft_ref, kit_ref, wqt_ref, wkt_ref,
               w1t_ref, w2t_ref, b1c_ref, bqr_ref, bkc_ref, b2r_ref,
               o_ref, kx_ref, wbd_ref):
    f32, bf16 = jnp.float32, jnp.bfloat16

    @pl.when(pl.program_id(0) == 0)
    def _build_kx():
        kmt = jax.lax.dot_general(wkt_ref[...], kit_ref[...], _DN1,
                                  preferred_element_type=f32)
        kx_ref[0:24, :] = kft_ref[...].astype(bf16)
        kx_ref[24:40, :] = (kmt + bkc_ref[...]).astype(bf16)
        kx_ref[40:57, :] = jnp.ones((17, NK), bf16)
        kx_ref[57:64, :] = jnp.zeros((7, NK), bf16)
        w2t = w2t_ref[...]
        row16 = jnp.concatenate([w2t] * 8, axis=1)
        tiled = jnp.concatenate([row16] * 8, axis=0)
        blk_mask = (
            jax.lax.broadcasted_iota(jnp.int32, (128, 256), 0) // 16
            == jax.lax.broadcasted_iota(jnp.int32, (128, 256), 1) // 32)
        wbd_ref[...] = jnp.where(blk_mask, tiled, 0.0).astype(bf16)

    eye24 = (jax.lax.broadcasted_iota(jnp.int32, (24, 24), 0) ==
             jax.lax.broadcasted_iota(jnp.int32, (24, 24), 1)
             ).astype(jnp.bfloat16)
    qf = jax.lax.dot_general(qet_ref[...].astype(bf16), eye24, _DN0,
                             preferred_element_type=f32)
    qm = jax.lax.dot_general(qit_ref[...], wqt_ref[...], _DN01,
                             preferred_element_type=f32)
    qm = qm + bqr_ref[...]
    a = jnp.concatenate(
        [qf, jnp.ones((TQ, D_MSG), f32), qm, jnp.ones((TQ, 1), f32),
         jnp.zeros((TQ, 7), f32)], axis=1)

    w1t = w1t_ref[...]
    w1d_t = w1t[:, 32:40]
    bt = jnp.concatenate(
        [w1d_t, w1d_t, w1d_t, w1t[:, 16:32], w1t[:, 0:16],
         b1c_ref[...], jnp.zeros((D_FF, 7), f32)], axis=1)

    qw = (a[:, None, :] * bt[None, :, :]).astype(bf16).reshape(
        TQ * D_FF, 64)
    h = jax.lax.dot_general(qw, kx_ref[...], _DN1,
                            preferred_element_type=f32)
    u = (h * 0.5).astype(bf16)
    t = jnp.tanh(u)
    sb = u * t + u

    b2_3d = b2r_ref[...][:, :, None]
    for g in range(TQ // 8):
        r = jax.lax.dot_general(wbd_ref[...], sb[g * 256:(g + 1) * 256, :],
                                _DN1, preferred_element_type=f32)
        o_ref[g * 8:(g + 1) * 8, :, :] = r.reshape(8, D_OUT, NK) + b2_3d


def kernel(q_equi, q_inv, k_equi, k_inv, Wq, bq, Wk, bk, W1, b1, W2, b2):
    f32 = jnp.float32
    qet = q_equi.reshape(NQ, 24).T
    qit = q_inv.reshape(NQ, -1).T
    kft = k_equi.reshape(NK, 24).T
    kit = k_inv.reshape(NK, -1).T

    out_t = pl.pallas_call(
        _pair_body,
        grid=(NQ // TQ,),
        in_specs=[
            pl.BlockSpec((24, TQ), lambda i: (0, i)),
            pl.BlockSpec((64, TQ), lambda i: (0, i)),
            pl.BlockSpec((24, NK), lambda i: (0, 0)),
            pl.BlockSpec((64, NK), lambda i: (0, 0)),
            pl.BlockSpec((D_MSG, 64), lambda i: (0, 0)),
            pl.BlockSpec((D_MSG, 64), lambda i: (0, 0)),
            pl.BlockSpec((D_FF, 40), lambda i: (0, 0)),
            pl.BlockSpec((D_MSG, D_FF), lambda i: (0, 0)),
            pl.BlockSpec((D_FF, 1), lambda i: (0, 0)),
            pl.BlockSpec((1, D_MSG), lambda i: (0, 0)),
            pl.BlockSpec((D_MSG, 1), lambda i: (0, 0)),
            pl.BlockSpec((1, D_OUT), lambda i: (0, 0)),
        ],
        out_specs=pl.BlockSpec((TQ, D_OUT, NK), lambda i: (i, 0, 0)),
        out_shape=jax.ShapeDtypeStruct((NQ, D_OUT, NK), f32),
        scratch_shapes=[pltpu.VMEM((64, NK), jnp.bfloat16),
                        pltpu.VMEM((128, 256), jnp.bfloat16)],
    )(qet, qit, kft, kit, Wq.T, Wk.T, W1.T, W2.T,
      b1[:, None], bq[None, :], bk[:, None], b2[None, :])

    return out_t.reshape(B, NQ, D_OUT, NK).transpose(0, 1, 3, 2)


# speedup vs baseline: 1.1145x; 1.0307x over previous
import jax
import jax.numpy as jnp
from jax.experimental import pallas as pl
from jax.experimental.pallas import tpu as pltpu

B, NQ, NK = 1, 2048, 1024
D_MSG, D_FF, D_OUT = 16, 32, 16
TQ = 128

_DN0 = (((0,), (0,)), ((), ()))
_DN1 = (((1,), (0,)), ((), ()))
_DN01 = (((0,), (1,)), ((), ()))


def _pair_body(qet_ref, qit_ref, kft_ref, kit_ref, wqt_ref, wkt_ref,
               w1t_ref, w2t_ref, b1c_ref, bqr_ref, bkc_ref, b2r_ref,
               o_ref, kx_ref, wbd_ref):
    f32, bf16 = jnp.float32, jnp.bfloat16

    @pl.when(pl.program_id(0) == 0)
    def _build_kx():
        kmt = jax.lax.dot_general(wkt_ref[...], kit_ref[...], _DN1,
                                  preferred_element_type=f32)
        kx_ref[0:24, :] = kft_ref[...].astype(bf16)
        kx_ref[24:40, :] = (kmt + bkc_ref[...]).astype(bf16)
        kx_ref[40:57, :] = jnp.ones((17, NK), bf16)
        kx_ref[57:64, :] = jnp.zeros((7, NK), bf16)
        w2t = w2t_ref[...]
        row16 = jnp.concatenate([w2t] * 8, axis=1)
        tiled = jnp.concatenate([row16] * 8, axis=0)
        blk_mask = (
            jax.lax.broadcasted_iota(jnp.int32, (128, 256), 0) // 16
            == jax.lax.broadcasted_iota(jnp.int32, (128, 256), 1) // 32)
        wbd_ref[...] = jnp.where(blk_mask, tiled, 0.0).astype(bf16)

    eye24 = (jax.lax.broadcasted_iota(jnp.int32, (24, 24), 0) ==
             jax.lax.broadcasted_iota(jnp.int32, (24, 24), 1)
             ).astype(jnp.bfloat16)
    qf = jax.lax.dot_general(qet_ref[...].astype(bf16), eye24, _DN0,
                             preferred_element_type=f32)
    qm = jax.lax.dot_general(qit_ref[...], wqt_ref[...], _DN01,
                             preferred_element_type=f32)
    qm = qm + bqr_ref[...]
    a = jnp.concatenate(
        [qf, jnp.ones((TQ, D_MSG), f32), qm, jnp.ones((TQ, 1), f32),
         jnp.zeros((TQ, 7), f32)], axis=1)

    w1t = w1t_ref[...]
    w1d_t = w1t[:, 32:40]
    bt = jnp.concatenate(
        [w1d_t, w1d_t, w1d_t, w1t[:, 16:32], w1t[:, 0:16],
         b1c_ref[...], jnp.zeros((D_FF, 7), f32)], axis=1)

    qw = (a[:, None, :] * bt[None, :, :]).astype(bf16).reshape(
        TQ * D_FF, 64)
    h = jax.lax.dot_general(qw, kx_ref[...], _DN1,
                            preferred_element_type=f32)
    u = h.astype(bf16) * jnp.bfloat16(0.5)
    t = jnp.tanh(u)
    sb = u * t + u

    b2_3d = b2r_ref[...][:, :, None]
    for g in range(TQ // 8):
        r = jax.lax.dot_general(wbd_ref[...], sb[g * 256:(g + 1) * 256, :],
                                _DN1, preferred_element_type=f32)
        o_ref[g * 8:(g + 1) * 8, :, :] = r.reshape(8, D_OUT, NK) + b2_3d


def kernel(q_equi, q_inv, k_equi, k_inv, Wq, bq, Wk, bk, W1, b1, W2, b2):
    f32 = jnp.float32
    qet = q_equi.reshape(NQ, 24).T
    qit = q_inv.reshape(NQ, -1).T
    kft = k_equi.reshape(NK, 24).T
    kit = k_inv.reshape(NK, -1).T

    out_t = pl.pallas_call(
        _pair_body,
        grid=(NQ // TQ,),
        in_specs=[
            pl.BlockSpec((24, TQ), lambda i: (0, i)),
            pl.BlockSpec((64, TQ), lambda i: (0, i)),
            pl.BlockSpec((24, NK), lambda i: (0, 0)),
            pl.BlockSpec((64, NK), lambda i: (0, 0)),
            pl.BlockSpec((D_MSG, 64), lambda i: (0, 0)),
            pl.BlockSpec((D_MSG, 64), lambda i: (0, 0)),
            pl.BlockSpec((D_FF, 40), lambda i: (0, 0)),
            pl.BlockSpec((D_MSG, D_FF), lambda i: (0, 0)),
            pl.BlockSpec((D_FF, 1), lambda i: (0, 0)),
            pl.BlockSpec((1, D_MSG), lambda i: (0, 0)),
            pl.BlockSpec((D_MSG, 1), lambda i: (0, 0)),
            pl.BlockSpec((1, D_OUT), lambda i: (0, 0)),
        ],
        out_specs=pl.BlockSpec((TQ, D_OUT, NK), lambda i: (i, 0, 0)),
        out_shape=jax.ShapeDtypeStruct((NQ, D_OUT, NK), f32),
        scratch_shapes=[pltpu.VMEM((64, NK), jnp.bfloat16),
                        pltpu.VMEM((128, 256), jnp.bfloat16)],
    )(qet, qit, kft, kit, Wq.T, Wk.T, W1.T, W2.T,
      b1[:, None], bq[None, :], bk[:, None], b2[None, :])

    return out_t.reshape(B, NQ, D_OUT, NK).transpose(0, 1, 3, 2)
